# speedup vs baseline: 2.6286x; 2.6286x over previous
import functools

import jax
import jax.numpy as jnp
from jax import lax
from jax.experimental import pallas as pl
from jax.experimental.pallas import tpu as pltpu
from jax.experimental.pallas import tpu_sc as plsc

_K = 8192
_D = 256
_MU = 0.25
_BT = 256

_NW = 32


def _vq_tc_body(x_ref, cbt_ref, probs_ref, idx_ref, loss_ref, cn_ref):
    @pl.when(pl.program_id(0) == 0)
    def _():
        cbt = cbt_ref[...]
        cn_ref[...] = jnp.sum(cbt * cbt, axis=0, keepdims=True)

    x = x_ref[...]
    xn = jnp.sum(x * x, axis=1, keepdims=True)
    dot = lax.dot_general(x, cbt_ref[...], (((1,), (0,)), ((), ())),
                          preferred_element_type=jnp.float32)
    d2 = (xn + cn_ref[...]) - 2.0 * dot
    logits = -d2
    m = jnp.max(logits, axis=1, keepdims=True)
    iota = lax.broadcasted_iota(jnp.int32, (_BT, _K), 1)
    idx = jnp.min(jnp.where(logits == m, iota, _K), axis=1)
    e = jnp.exp(logits - m)
    s = jnp.sum(e, axis=1, keepdims=True)
    probs_ref[...] = e / s
    idx_ref[0, 0, :] = idx
    loss_ref[0, 0, :] = -m[:, 0] * ((1.0 + _MU) / _D)


def _vq_distance_softmax(xf, cbt):
    n = xf.shape[0]
    nb = n // _BT
    return pl.pallas_call(
        _vq_tc_body,
        grid=(nb,),
        in_specs=[
            pl.BlockSpec((_BT, _D), lambda i: (i, 0)),
            pl.BlockSpec((_D, _K), lambda i: (0, 0)),
        ],
        out_specs=[
            pl.BlockSpec((_BT, _K), lambda i: (i, 0)),
            pl.BlockSpec((1, 1, _BT), lambda i: (i, 0, 0)),
            pl.BlockSpec((1, 1, _BT), lambda i: (i, 0, 0)),
        ],
        out_shape=[
            jax.ShapeDtypeStruct((n, _K), jnp.float32),
            jax.ShapeDtypeStruct((nb, 1, _BT), jnp.int32),
            jax.ShapeDtypeStruct((nb, 1, _BT), jnp.float32),
        ],
        scratch_shapes=[pltpu.VMEM((1, _K), jnp.float32)],
        compiler_params=pltpu.CompilerParams(
            dimension_semantics=("arbitrary",),
        ),
    )(xf, cbt)


def _sc_gather(codebook, idx_flat):
    n = idx_flat.shape[0]
    bpw = n // _NW
    mesh = plsc.VectorSubcoreMesh(core_axis_name="c", subcore_axis_name="s")

    @functools.partial(
        pl.kernel,
        mesh=mesh,
        out_type=jax.ShapeDtypeStruct((n, _D), jnp.float32),
        scratch_types=[
            pltpu.VMEM((bpw,), jnp.int32),
            pltpu.VMEM((bpw, _D), jnp.float32),
            pltpu.SemaphoreType.DMA,
        ],
    )
    def k(table_hbm, idx_hbm, out_hbm, idx_v, rows_v, sem):
        wid = lax.axis_index("s") * 2 + lax.axis_index("c")
        base = wid * bpw
        pltpu.sync_copy(idx_hbm.at[pl.ds(base, bpw)], idx_v)
        pltpu.async_copy(table_hbm.at[idx_v], rows_v, sem).wait()
        pltpu.sync_copy(rows_v, out_hbm.at[pl.ds(base, bpw)])

    return k(codebook, idx_flat)


def kernel(x, codebook):
    b, t, d = x.shape
    xf = x.reshape(-1, d)
    cbt = codebook.T
    probs, idx3, loss3 = _vq_distance_softmax(xf, cbt)
    idx_flat = idx3.reshape(-1)
    quants = _sc_gather(codebook, idx_flat)
    k = codebook.shape[0]
    return (idx_flat.reshape(b, t),
            probs.reshape(b, t, k),
            quants.reshape(b, t, d),
            loss3.reshape(b, t))


# speedup vs baseline: 2.6671x; 1.0147x over previous
import functools

import jax
import jax.numpy as jnp
from jax import lax
from jax.experimental import pallas as pl
from jax.experimental.pallas import tpu as pltpu
from jax.experimental.pallas import tpu_sc as plsc

_K = 8192
_D = 256
_MU = 0.25
_BT = 256

_NW = 32


def _cn_body(cbt_ref, cn_ref):
    cbt = cbt_ref[...]
    cn_ref[...] = jnp.sum(cbt * cbt, axis=0, keepdims=True)


def _code_norms(cbt):
    return pl.pallas_call(
        _cn_body,
        out_shape=jax.ShapeDtypeStruct((1, _K), jnp.float32),
    )(cbt)


def _vq_tc_body(x_ref, cbt_ref, cn_ref, probs_ref, idx_ref, loss_ref):
    x = x_ref[...]
    xn = jnp.sum(x * x, axis=1, keepdims=True)
    dot = lax.dot_general(x, cbt_ref[...], (((1,), (0,)), ((), ())),
                          preferred_element_type=jnp.float32)
    a = xn + cn_ref[...]
    logits = 2.0 * dot - a
    m = jnp.max(logits, axis=1, keepdims=True)
    idx = jnp.argmax(logits, axis=1)
    e = jnp.exp(logits - m)
    s = jnp.sum(e, axis=1, keepdims=True)
    probs_ref[...] = e * (1.0 / s)
    idx_ref[0, 0, :] = idx
    loss_ref[0, 0, :] = m[:, 0] * (-(1.0 + _MU) / _D)


def _vq_distance_softmax(xf, cbt, cn):
    n = xf.shape[0]
    nb = n // _BT
    return pl.pallas_call(
        _vq_tc_body,
        grid=(nb,),
        in_specs=[
            pl.BlockSpec((_BT, _D), lambda i: (i, 0)),
            pl.BlockSpec((_D, _K), lambda i: (0, 0)),
            pl.BlockSpec((1, _K), lambda i: (0, 0)),
        ],
        out_specs=[
            pl.BlockSpec((_BT, _K), lambda i: (i, 0)),
            pl.BlockSpec((1, 1, _BT), lambda i: (i, 0, 0)),
            pl.BlockSpec((1, 1, _BT), lambda i: (i, 0, 0)),
        ],
        out_shape=[
            jax.ShapeDtypeStruct((n, _K), jnp.float32),
            jax.ShapeDtypeStruct((nb, 1, _BT), jnp.int32),
            jax.ShapeDtypeStruct((nb, 1, _BT), jnp.float32),
        ],
        compiler_params=pltpu.CompilerParams(
            dimension_semantics=("arbitrary",),
        ),
    )(xf, cbt, cn)


def _sc_gather(codebook, idx_flat):
    n = idx_flat.shape[0]
    bpw = n // _NW
    mesh = plsc.VectorSubcoreMesh(core_axis_name="c", subcore_axis_name="s")

    @functools.partial(
        pl.kernel,
        mesh=mesh,
        out_type=jax.ShapeDtypeStruct((n, _D), jnp.float32),
        scratch_types=[
            pltpu.VMEM((bpw,), jnp.int32),
            pltpu.VMEM((bpw, _D), jnp.float32),
            pltpu.SemaphoreType.DMA,
        ],
    )
    def k(table_hbm, idx_hbm, out_hbm, idx_v, rows_v, sem):
        wid = lax.axis_index("s") * 2 + lax.axis_index("c")
        base = wid * bpw
        pltpu.sync_copy(idx_hbm.at[pl.ds(base, bpw)], idx_v)
        pltpu.async_copy(table_hbm.at[idx_v], rows_v, sem).wait()
        pltpu.sync_copy(rows_v, out_hbm.at[pl.ds(base, bpw)])

    return k(codebook, idx_flat)


def kernel(x, codebook):
    b, t, d = x.shape
    xf = x.reshape(-1, d)
    cbt = codebook.T
    cn = _code_norms(cbt)
    probs, idx3, loss3 = _vq_distance_softmax(xf, cbt, cn)
    idx_flat = idx3.reshape(-1)
    quants = _sc_gather(codebook, idx_flat)
    k = codebook.shape[0]
    return (idx_flat.reshape(b, t),
            probs.reshape(b, t, k),
            quants.reshape(b, t, d),
            loss3.reshape(b, t))


# speedup vs baseline: 2.7503x; 1.0312x over previous
import functools

import jax
import jax.numpy as jnp
from jax import lax
from jax.experimental import pallas as pl
from jax.experimental.pallas import tpu as pltpu
from jax.experimental.pallas import tpu_sc as plsc

_K = 8192
_D = 256
_MU = 0.25
_BT = 256

_NW = 32


def _cn_body(cb_ref, cn_ref):
    cbt = cb_ref[...].T
    cn_ref[...] = jnp.sum(cbt * cbt, axis=0, keepdims=True)


def _code_norms(cb):
    return pl.pallas_call(
        _cn_body,
        out_shape=jax.ShapeDtypeStruct((1, _K), jnp.float32),
    )(cb)


def _vq_tc_body(x_ref, cb_ref, cn_ref, probs_ref, idx_ref, loss_ref):
    x = x_ref[...]
    xn = jnp.sum(x * x, axis=1, keepdims=True)
    dot = lax.dot_general(x, cb_ref[...], (((1,), (1,)), ((), ())),
                          preferred_element_type=jnp.float32)
    a = xn + cn_ref[...]
    logits = 2.0 * dot - a
    m = jnp.max(logits, axis=1, keepdims=True)
    idx = jnp.argmax(logits, axis=1)
    e = jnp.exp(logits - m)
    s = jnp.sum(e, axis=1, keepdims=True)
    probs_ref[...] = e * (1.0 / s)
    idx_ref[0, 0, :] = idx
    loss_ref[0, 0, :] = m[:, 0] * (-(1.0 + _MU) / _D)


def _vq_distance_softmax(xf, cb, cn):
    n = xf.shape[0]
    nb = n // _BT
    return pl.pallas_call(
        _vq_tc_body,
        grid=(nb,),
        in_specs=[
            pl.BlockSpec((_BT, _D), lambda i: (i, 0)),
            pl.BlockSpec((_K, _D), lambda i: (0, 0)),
            pl.BlockSpec((1, _K), lambda i: (0, 0)),
        ],
        out_specs=[
            pl.BlockSpec((_BT, _K), lambda i: (i, 0)),
            pl.BlockSpec((1, 1, _BT), lambda i: (i, 0, 0)),
            pl.BlockSpec((1, 1, _BT), lambda i: (i, 0, 0)),
        ],
        out_shape=[
            jax.ShapeDtypeStruct((n, _K), jnp.float32),
            jax.ShapeDtypeStruct((nb, 1, _BT), jnp.int32),
            jax.ShapeDtypeStruct((nb, 1, _BT), jnp.float32),
        ],
        compiler_params=pltpu.CompilerParams(
            dimension_semantics=("arbitrary",),
        ),
    )(xf, cb, cn)


def _sc_gather(codebook, idx_flat):
    n = idx_flat.shape[0]
    bpw = n // _NW
    mesh = plsc.VectorSubcoreMesh(core_axis_name="c", subcore_axis_name="s")

    @functools.partial(
        pl.kernel,
        mesh=mesh,
        out_type=jax.ShapeDtypeStruct((n, _D), jnp.float32),
        scratch_types=[
            pltpu.VMEM((bpw,), jnp.int32),
            pltpu.VMEM((bpw, _D), jnp.float32),
            pltpu.SemaphoreType.DMA,
        ],
    )
    def k(table_hbm, idx_hbm, out_hbm, idx_v, rows_v, sem):
        wid = lax.axis_index("s") * 2 + lax.axis_index("c")
        base = wid * bpw
        pltpu.sync_copy(idx_hbm.at[pl.ds(base, bpw)], idx_v)
        pltpu.async_copy(table_hbm.at[idx_v], rows_v, sem).wait()
        pltpu.sync_copy(rows_v, out_hbm.at[pl.ds(base, bpw)])

    return k(codebook, idx_flat)


def kernel(x, codebook):
    b, t, d = x.shape
    xf = x.reshape(-1, d)
    cn = _code_norms(codebook)
    probs, idx3, loss3 = _vq_distance_softmax(xf, codebook, cn)
    idx_flat = idx3.reshape(-1)
    quants = _sc_gather(codebook, idx_flat)
    k = codebook.shape[0]
    return (idx_flat.reshape(b, t),
            probs.reshape(b, t, k),
            quants.reshape(b, t, d),
            loss3.reshape(b, t))
